# Initial kernel scaffold; baseline (speedup 1.0000x reference)
import jax
import jax.numpy as jnp
from jax.experimental import pallas as pl


def kernel(x, W):
    raise NotImplementedError("write your pallas kernel here")


# speedup vs baseline: 1.7522x; 1.7522x over previous
import functools

import jax
import jax.numpy as jnp
from jax import lax
from jax.experimental import pallas as pl
from jax.experimental.pallas import tpu as pltpu
from jax.experimental.pallas import tpu_sc as plsc

_NUM_CORES = 2
_NUM_SUBCORES = 16
_NW = _NUM_CORES * _NUM_SUBCORES
_CHUNK = 128


def _argmin_body(x_ref, w_ref, idx_ref):
    xb = x_ref[0]
    w = w_ref[...]
    s = lax.dot_general(w, xb, (((1,), (0,)), ((), ())),
                        preferred_element_type=jnp.float32)
    b2 = jnp.sum(w * w, axis=1, keepdims=True)
    a2 = jnp.sum(xb * xb, axis=0, keepdims=True)
    d2 = jnp.maximum(a2 + b2 - 2.0 * s, 0.0)
    dist = jnp.sqrt(d2)
    idx_ref[0, 0, :] = jnp.argmin(dist, axis=0).astype(jnp.int32)


def _nearest_indices(x_r, w):
    b, c, n = x_r.shape
    k = w.shape[0]
    return pl.pallas_call(
        _argmin_body,
        grid=(b,),
        in_specs=[
            pl.BlockSpec((1, c, n), lambda i: (i, 0, 0)),
            pl.BlockSpec((k, c), lambda i: (0, 0)),
        ],
        out_specs=pl.BlockSpec((1, 1, n), lambda i: (i, 0, 0)),
        out_shape=jax.ShapeDtypeStruct((b, 1, n), jnp.int32),
    )(x_r, w)


def _make_sc_gather(rows, d):
    per_w = rows // _NW
    n_ch = per_w // _CHUNK
    mesh = plsc.VectorSubcoreMesh(core_axis_name="c", subcore_axis_name="s")

    @functools.partial(
        pl.kernel,
        mesh=mesh,
        out_type=jax.ShapeDtypeStruct((rows, d), jnp.float32),
        compiler_params=pltpu.CompilerParams(use_tc_tiling_on_sc=False),
        scratch_types=[
            pltpu.VMEM((n_ch, _CHUNK), jnp.int32),
            pltpu.VMEM((per_w, d), jnp.float32),
            pltpu.SemaphoreType.DMA,
        ],
    )
    def gather_kernel(table_hbm, idx_hbm, out_hbm, idx_v, rows_v, sem):
        wid = lax.axis_index("s") * _NUM_CORES + lax.axis_index("c")
        base = wid * per_w
        pltpu.sync_copy(idx_hbm.at[wid], idx_v)
        copies = []
        for j in range(n_ch):
            copies.append(pltpu.async_copy(
                table_hbm.at[idx_v.at[j]],
                rows_v.at[pl.ds(j * _CHUNK, _CHUNK)],
                sem,
            ))
        for cp in copies:
            cp.wait()
        pltpu.sync_copy(rows_v, out_hbm.at[pl.ds(base, per_w)])

    return gather_kernel


def kernel(x, W):
    b, c, h, w_sp = x.shape
    n = h * w_sp
    rows = b * n
    x_r = x.reshape(b, c, n)
    idx = _nearest_indices(x_r, W)
    idx3 = idx.reshape(_NW, rows // _NW // _CHUNK, _CHUNK)
    quantized = _make_sc_gather(rows, c)(W, idx3)
    return quantized.reshape(x.shape)
